# Initial kernel scaffold; baseline (speedup 1.0000x reference)
import jax
import jax.numpy as jnp
from jax.experimental import pallas as pl


def kernel(x1, edge_index1, edge_attr1, x2, edge_index2, edge_attr2, label, c1_W1, c1_b1, c1_W2, c1_b2, c1_root, c1_bias, c2_W1, c2_b1, c2_W2, c2_b2, c2_root, c2_bias, fc_W, fc_b):
    raise NotImplementedError("write your pallas kernel here")


# speedup vs baseline: 2.5701x; 2.5701x over previous
import functools

import jax
import jax.numpy as jnp
from jax import lax
from jax.experimental import pallas as pl
from jax.experimental.pallas import tpu as pltpu
from jax.experimental.pallas import tpu_sc as plsc

N = 10000
E = 160000
DIM = 16
B = 4096
NC = 2
NS = 16
NW = NC * NS
L = 16
CHUNK = 128


def _sc_mesh():
    return plsc.VectorSubcoreMesh(core_axis_name="c", subcore_axis_name="s")


def _chunk_of(b_per_w):
    for c in range(min(b_per_w, CHUNK), 0, -1):
        if b_per_w % c == 0:
            return c
    return 1


def _sc_gather(table, idx):
    nrows = idx.shape[0]
    b_per_w = nrows // NW
    chunk = _chunk_of(b_per_w)
    n_chunks = b_per_w // chunk
    idx2 = idx.reshape(NW, n_chunks, chunk)

    @functools.partial(
        pl.kernel,
        mesh=_sc_mesh(),
        out_type=jax.ShapeDtypeStruct((nrows, DIM), jnp.float32),
        scratch_types=[
            pltpu.VMEM((n_chunks, chunk), jnp.int32),
            pltpu.VMEM((b_per_w, DIM), jnp.float32),
            pltpu.SemaphoreType.DMA,
        ],
        compiler_params=pltpu.CompilerParams(use_tc_tiling_on_sc=False),
    )
    def k(table_hbm, idx_hbm, out_hbm, idx_v, rows_v, sem):
        wid = lax.axis_index("s") * NC + lax.axis_index("c")
        pltpu.sync_copy(idx_hbm.at[wid], idx_v)

        def fire(j):
            return pltpu.async_copy(
                table_hbm.at[idx_v.at[j]],
                rows_v.at[pl.ds(j * chunk, chunk)],
                sem,
            )

        def body(g, _):
            cps = [fire(g * 8 + b) for b in range(8)]
            for cp in cps:
                cp.wait()
            return 0

        lax.fori_loop(0, n_chunks // 8, body, 0, unroll=False)
        rem = n_chunks % 8
        if rem:
            cps = [fire((n_chunks // 8) * 8 + b) for b in range(rem)]
            for cp in cps:
                cp.wait()
        pltpu.sync_copy(rows_v, out_hbm.at[pl.ds(wid * b_per_w, b_per_w)])

    return k(table, idx2)


def _sc_scatter(msg, dst):
    e_per_w = E // NW
    chunk = _chunk_of(e_per_w)
    n_chunks = e_per_w // chunk
    rows_per_s = N // NS
    dst2 = dst.reshape(NW, n_chunks, chunk)

    @functools.partial(
        pl.kernel,
        mesh=_sc_mesh(),
        out_type=jax.ShapeDtypeStruct((NC, N, DIM), jnp.float32),
        scratch_types=[
            pltpu.VMEM((n_chunks, chunk), jnp.int32),
            pltpu.VMEM((e_per_w, DIM), jnp.float32),
            pltpu.VMEM((rows_per_s, DIM), jnp.float32),
            pltpu.VMEM_SHARED((N, DIM), jnp.float32),
        ],
        compiler_params=pltpu.CompilerParams(use_tc_tiling_on_sc=False),
    )
    def k(msg_hbm, dst_hbm, out_hbm, dst_v, msg_v, zbuf, acc_sh):
        c = lax.axis_index("c")
        s = lax.axis_index("s")
        wid = s * NC + c

        def zloop(i, _):
            zbuf[i, :] = jnp.zeros((L,), jnp.float32)
            return 0

        lax.fori_loop(0, rows_per_s, zloop, 0)
        pltpu.sync_copy(zbuf, acc_sh.at[pl.ds(s * rows_per_s, rows_per_s)])
        pltpu.sync_copy(dst_hbm.at[wid], dst_v)
        pltpu.sync_copy(msg_hbm.at[pl.ds(wid * e_per_w, e_per_w)], msg_v)
        plsc.subcore_barrier()

        def body(j, _):
            pltpu.sync_copy(
                msg_v.at[pl.ds(j * chunk, chunk)],
                acc_sh.at[dst_v.at[j]],
                add=True,
            )
            return 0

        lax.fori_loop(0, n_chunks, body, 0, unroll=False)
        plsc.subcore_barrier()
        pltpu.sync_copy(
            acc_sh.at[pl.ds(s * rows_per_s, rows_per_s)],
            out_hbm.at[c, pl.ds(s * rows_per_s, rows_per_s)],
        )

    return k(msg, dst2)


def _msg_body(ea_ref, xj_ref, w1_ref, b1_ref, w2p_ref, b2r_ref, ri_ref, out_ref):
    ea = ea_ref[...]
    xj = xj_ref[...]
    h = jnp.maximum(
        jnp.dot(ea, w1_ref[...], preferred_element_type=jnp.float32) + b1_ref[...],
        0.0,
    )
    xj_exp = jnp.dot(xj, ri_ref[...], preferred_element_type=jnp.float32)
    z = xj_exp * jnp.tile(h, (1, L))
    out_ref[...] = (
        jnp.dot(z, w2p_ref[...], preferred_element_type=jnp.float32)
        + jnp.dot(xj, b2r_ref[...], preferred_element_type=jnp.float32)
    )


def _msg_pallas(ea, xj, w1, b1, w2p, b2r, ri):
    TE = 2000
    grid = E // TE
    return pl.pallas_call(
        _msg_body,
        grid=(grid,),
        in_specs=[
            pl.BlockSpec((TE, 2), lambda i: (i, 0)),
            pl.BlockSpec((TE, DIM), lambda i: (i, 0)),
            pl.BlockSpec((2, DIM), lambda i: (0, 0)),
            pl.BlockSpec((1, DIM), lambda i: (0, 0)),
            pl.BlockSpec((DIM * DIM, DIM), lambda i: (0, 0)),
            pl.BlockSpec((DIM, DIM), lambda i: (0, 0)),
            pl.BlockSpec((DIM, DIM * DIM), lambda i: (0, 0)),
        ],
        out_specs=pl.BlockSpec((TE, DIM), lambda i: (i, 0)),
        out_shape=jax.ShapeDtypeStruct((E, DIM), jnp.float32),
    )(ea, xj, w1, b1, w2p, b2r, ri)


def _epi_body(p0_ref, p1_ref, x_ref, root_ref, bias_ref, out_ref):
    agg = p0_ref[...] + p1_ref[...]
    xr = jnp.dot(x_ref[...], root_ref[...], preferred_element_type=jnp.float32)
    out_ref[...] = jnp.maximum(agg + xr + bias_ref[...], 0.0)


def _epi_pallas(p0, p1, x, root, bias):
    TN = 2000
    grid = N // TN
    return pl.pallas_call(
        _epi_body,
        grid=(grid,),
        in_specs=[
            pl.BlockSpec((TN, DIM), lambda i: (i, 0)),
            pl.BlockSpec((TN, DIM), lambda i: (i, 0)),
            pl.BlockSpec((TN, DIM), lambda i: (i, 0)),
            pl.BlockSpec((DIM, DIM), lambda i: (0, 0)),
            pl.BlockSpec((1, DIM), lambda i: (0, 0)),
        ],
        out_specs=pl.BlockSpec((TN, DIM), lambda i: (i, 0)),
        out_shape=jax.ShapeDtypeStruct((N, DIM), jnp.float32),
    )(p0, p1, x, root, bias)


def _fc_body(fl_ref, fr_ref, wl_ref, wr_ref, b_ref, out_ref):
    acc = (
        jnp.dot(fl_ref[...], wl_ref[...], preferred_element_type=jnp.float32)
        + jnp.dot(fr_ref[...], wr_ref[...], preferred_element_type=jnp.float32)
        + b_ref[...]
    )
    out_ref[...] = jnp.maximum(acc, 0.0)


def _fc_pallas(fl, fr, wl, wr, b):
    return pl.pallas_call(
        _fc_body,
        grid=(1,),
        in_specs=[
            pl.BlockSpec((B, DIM), lambda i: (0, 0)),
            pl.BlockSpec((B, DIM), lambda i: (0, 0)),
            pl.BlockSpec((DIM, DIM), lambda i: (0, 0)),
            pl.BlockSpec((DIM, DIM), lambda i: (0, 0)),
            pl.BlockSpec((1, DIM), lambda i: (0, 0)),
        ],
        out_specs=pl.BlockSpec((B, DIM), lambda i: (0, 0)),
        out_shape=jax.ShapeDtypeStruct((B, DIM), jnp.float32),
    )(fl, fr, wl, wr, b)


def _prep_conv(w1, b1, w2, b2):
    w2p = jnp.transpose(w2.reshape(DIM, DIM, DIM), (1, 0, 2)).reshape(DIM * DIM, DIM)
    b2r = b2.reshape(DIM, DIM)
    return w1, b1.reshape(1, DIM), w2p, b2r


def _conv(x, src, dst, ea, params, ri):
    w1, b1, w2p, b2r, root, bias = params
    xj = _sc_gather(x, src)
    msg = _msg_pallas(ea, xj, w1, b1, w2p, b2r, ri)
    parts = _sc_scatter(msg, dst)
    return _epi_pallas(parts[0], parts[1], x, root, bias)


def kernel(x1, edge_index1, edge_attr1, x2, edge_index2, edge_attr2, label,
           c1_W1, c1_b1, c1_W2, c1_b2, c1_root, c1_bias,
           c2_W1, c2_b1, c2_W2, c2_b2, c2_root, c2_bias,
           fc_W, fc_b):
    ri = jnp.kron(jnp.eye(DIM, dtype=jnp.float32), jnp.ones((1, L), jnp.float32))
    w1a, b1a, w2pa, b2ra = _prep_conv(c1_W1, c1_b1, c1_W2, c1_b2)
    w1b, b1b, w2pb, b2rb = _prep_conv(c2_W1, c2_b1, c2_W2, c2_b2)
    p1 = (w1a, b1a, w2pa, b2ra, c1_root, c1_bias.reshape(1, DIM))
    p2 = (w1b, b1b, w2pb, b2rb, c2_root, c2_bias.reshape(1, DIM))

    def gcn(x, edge_index, edge_attr):
        ea = edge_attr.reshape(E, 2)
        src, dst = edge_index[0], edge_index[1]
        y = _conv(x, src, dst, ea, p1, ri)
        return _conv(y, src, dst, ea, p2, ri)

    x_lig = gcn(x1, edge_index1, edge_attr1)
    x_rec = gcn(x2, edge_index2, edge_attr2)
    fl = _sc_gather(x_lig, label[:, 0])
    fr = _sc_gather(x_rec, label[:, 1])
    return _fc_pallas(fl, fr, fc_W[:DIM], fc_W[DIM:], fc_b.reshape(1, DIM))
